# Initial kernel scaffold; baseline (speedup 1.0000x reference)
import jax
import jax.numpy as jnp
from jax import lax
from jax.experimental import pallas as pl
from jax.experimental.pallas import tpu as pltpu
from jax.experimental.pallas import tpu_sc as plsc

_DIM = 32
_LANE = 128
_NB = 8
_CHUNK = _NB * _LANE
_NW = 32


def _gather_body(table_hbm, idx_hbm, out_hbm, idx_v, rows_v, gsem):
    n_rows = idx_hbm.shape[0]
    rows_w = n_rows // _NW
    n_chunk = rows_w // _NB
    wid = lax.axis_index("s") * 2 + lax.axis_index("c")
    row0 = wid * rows_w

    def body(g, carry):
        irow = pl.multiple_of(row0 + g * _NB, _NB)
        pltpu.sync_copy(idx_hbm.at[pl.ds(irow, _NB)], idx_v)
        copies = []
        for j in range(_NB):
            copies.append(
                pltpu.async_copy(
                    table_hbm.at[idx_v.at[j]],
                    rows_v.at[pl.ds(j * _LANE, _LANE)],
                    gsem,
                )
            )
        for c in copies:
            c.wait()
        out0 = pl.multiple_of(irow * _LANE, _CHUNK)
        pltpu.sync_copy(rows_v, out_hbm.at[pl.ds(out0, _CHUNK)])
        return carry

    lax.fori_loop(0, n_chunk, body, 0)


@jax.jit
def _sc_gather(idx2d, weight):
    n_rows = idx2d.shape[0]
    mesh = plsc.VectorSubcoreMesh(core_axis_name="c", subcore_axis_name="s")
    return pl.kernel(
        _gather_body,
        out_type=jax.ShapeDtypeStruct((n_rows * _LANE, _DIM), jnp.float32),
        mesh=mesh,
        scratch_types=[
            pltpu.VMEM((_NB, _LANE), jnp.int32),
            pltpu.VMEM((_CHUNK, _DIM), jnp.float32),
            pltpu.SemaphoreType.DMA,
        ],
    )(weight, idx2d)


def kernel(indices, weight):
    b, s = indices.shape
    idx2d = indices.astype(jnp.int32).reshape(b * s // _LANE, _LANE)
    out = _sc_gather(idx2d, weight)
    return out.reshape(b, s, _DIM)


# speedup vs baseline: 1.0934x; 1.0934x over previous
import jax
import jax.numpy as jnp
from jax import lax
from jax.experimental import pallas as pl
from jax.experimental.pallas import tpu as pltpu
from jax.experimental.pallas import tpu_sc as plsc

_DIM = 32
_LANE = 128
_NB = 8
_CHUNK = _NB * _LANE
_NW = 32


def _gather_body(table_hbm, idx_hbm, out_hbm, idx_v, rows_v, gsem):
    n_rows = idx_hbm.shape[0]
    rows_w = n_rows // _NW
    n_chunk = rows_w // _NB
    wid = lax.axis_index("s") * 2 + lax.axis_index("c")
    row0 = wid * rows_w

    def body(g, carry):
        irow = pl.multiple_of(row0 + g * _NB, _NB)
        pltpu.sync_copy(idx_hbm.at[pl.ds(irow, _NB)], idx_v)
        copies = []
        for j in range(_NB):
            copies.append(
                pltpu.async_copy(
                    table_hbm.at[idx_v.at[j]],
                    rows_v.at[pl.ds(j * _LANE, _LANE)],
                    gsem,
                )
            )
        for c in copies:
            c.wait()
        out0 = pl.multiple_of(irow * _LANE, _CHUNK)
        pltpu.sync_copy(rows_v, out_hbm.at[pl.ds(out0, _CHUNK)])
        return carry

    lax.fori_loop(0, n_chunk, body, 0)


@jax.jit
def _sc_gather(idx2d, weight):
    n_rows = idx2d.shape[0]
    mesh = plsc.VectorSubcoreMesh(core_axis_name="c", subcore_axis_name="s")
    return pl.kernel(
        _gather_body,
        out_type=jax.ShapeDtypeStruct((n_rows * _LANE, _DIM), jnp.float32),
        mesh=mesh,
        scratch_types=[
            pltpu.VMEM((_NB, _LANE), jnp.int32),
            pltpu.VMEM((_CHUNK, _DIM), jnp.float32),
            pltpu.SemaphoreType.DMA,
        ],
        compiler_params=pltpu.CompilerParams(use_tc_tiling_on_sc=False),
    )(weight, idx2d)


def kernel(indices, weight):
    b, s = indices.shape
    idx2d = indices.astype(jnp.int32).reshape(b * s // _LANE, _LANE)
    out = _sc_gather(idx2d, weight)
    return out.reshape(b, s, _DIM)


# speedup vs baseline: 1.7398x; 1.5911x over previous
import jax
import jax.numpy as jnp
from jax import lax
from jax.experimental import pallas as pl
from jax.experimental.pallas import tpu as pltpu
from jax.experimental.pallas import tpu_sc as plsc

_DIM = 32
_SEQ = 50
_NB = 16
_NW = 32


def _gather_body(table_hbm, idx_hbm, out_hbm, idx_v, rows_v, gsem):
    n_batch = idx_hbm.shape[0]
    bat_w = n_batch // _NW
    n_chunk = bat_w // _NB
    wid = lax.axis_index("s") * 2 + lax.axis_index("c")
    b_w = wid * bat_w

    def body(g, carry):
        b0 = pl.multiple_of(b_w + g * _NB, _NB)
        pltpu.sync_copy(idx_hbm.at[pl.ds(b0, _NB)], idx_v)
        copies = []
        for j in range(_NB):
            copies.append(
                pltpu.async_copy(
                    table_hbm.at[idx_v.at[j]],
                    rows_v.at[j],
                    gsem,
                )
            )
        for c in copies:
            c.wait()
        pltpu.sync_copy(rows_v, out_hbm.at[pl.ds(b0, _NB)])
        return carry

    lax.fori_loop(0, n_chunk, body, 0)


@jax.jit
def _sc_gather(indices, weight):
    n_batch, seq = indices.shape
    mesh = plsc.VectorSubcoreMesh(core_axis_name="c", subcore_axis_name="s")
    return pl.kernel(
        _gather_body,
        out_type=jax.ShapeDtypeStruct((n_batch, seq, _DIM), jnp.float32),
        mesh=mesh,
        scratch_types=[
            pltpu.VMEM((_NB, _SEQ), jnp.int32),
            pltpu.VMEM((_NB, _SEQ, _DIM), jnp.float32),
            pltpu.SemaphoreType.DMA,
        ],
        compiler_params=pltpu.CompilerParams(use_tc_tiling_on_sc=False),
    )(weight, indices)


def kernel(indices, weight):
    return _sc_gather(indices.astype(jnp.int32), weight)
